# Initial kernel scaffold; baseline (speedup 1.0000x reference)
import jax
import jax.numpy as jnp
from jax.experimental import pallas as pl


def kernel(positions, cell, edge_shifts, numbers, edge_index):
    raise NotImplementedError("write your pallas kernel here")


# speedup vs baseline: 1.3375x; 1.3375x over previous
import functools

import jax
import jax.numpy as jnp
import numpy as np
from jax.experimental import pallas as pl

_ALL_SPECIES = np.array([1, 6, 7, 8], dtype=np.int32)
_N_SPECIES = 4
_R_CUT = 5.0
_AVG_NEIGH = 40.0
_Z = [
    np.array([3.14159265, 6.28318531, 9.42477796, 12.56637061, 15.70796327,
              18.84955592, 21.99114858, 25.13274123]),
    np.array([4.49340946, 7.72525184, 10.90412166, 14.06619391, 17.22075527,
              20.37130296, 23.51945250]),
    np.array([5.76345920, 9.09501133, 12.32294097, 15.51460301, 18.68903640,
              21.85387410]),
    np.array([6.98793200, 10.41711855, 13.69802315, 16.92362129, 20.12180617]),
]
_NL = [8, 7, 6, 5]
_NF = 26


def _feat_body(zcol_ref, d2_ref, out_ref):
    d2 = d2_ref[0]
    r = jnp.sqrt(d2 + 1e-12)
    x = zcol_ref[...] * r / _R_CUT
    s, c = jnp.sin(x), jnp.cos(x)
    row = 0
    for l in range(4):
        sl = slice(row, row + _NL[l])
        xs, ss, cc = x[sl], s[sl], c[sl]
        if l == 0:
            rb = ss / xs
        elif l == 1:
            rb = ss / (xs * xs) - cc / xs
        elif l == 2:
            x2 = xs * xs
            x3 = x2 * xs
            rb = (3.0 / x3 - 1.0 / xs) * ss - 3.0 * cc / x2
        else:
            x2 = xs * xs
            x3 = x2 * xs
            x4 = x2 * x2
            rb = (15.0 / x4 - 6.0 / x2) * ss - (15.0 / x3 - 1.0 / xs) * cc
        out_ref[sl, :] = rb
        row += _NL[l]


def _features_planes(d2):
    E = d2.shape[0]
    BF = 2560
    assert E % BF == 0
    g = E // BF
    d2v = d2.reshape(g, 1, BF)
    zcol = jnp.asarray(np.concatenate(_Z).astype(np.float32).reshape(_NF, 1))
    return pl.pallas_call(
        _feat_body,
        grid=(g,),
        in_specs=[
            pl.BlockSpec((_NF, 1), lambda i: (0, 0)),
            pl.BlockSpec((1, 1, BF), lambda i: (i, 0, 0)),
        ],
        out_specs=pl.BlockSpec((_NF, BF), lambda i: (0, i)),
        out_shape=jax.ShapeDtypeStruct((_NF, E), jnp.float32),
    )(zcol, d2v)


def kernel(positions, cell, edge_shifts, numbers, edge_index):
    N = positions.shape[0]
    e0 = edge_index[0]
    e1 = edge_index[1]
    vec = positions[e1] - positions[e0] + edge_shifts @ cell
    d2 = jnp.sum(vec * vec, axis=-1)
    feats = _features_planes(d2)

    species_idx = jnp.searchsorted(jnp.asarray(_ALL_SPECIES), numbers)
    dens_idx = e0 * _N_SPECIES + species_idx[e1]
    dens = jnp.zeros((_NF, N * _N_SPECIES), jnp.float32).at[:, dens_idx].add(feats)
    norms = np.concatenate([
        np.full(_NL[l], (1.0 / _AVG_NEIGH ** 0.75) if l == 0 else (1.0 / np.sqrt(_AVG_NEIGH)))
        for l in range(4)
    ]).astype(np.float32)
    dens = dens * jnp.asarray(norms)[:, None]
    out = dens.reshape(_NF, N, _N_SPECIES).transpose(1, 0, 2).reshape(N, _NF * _N_SPECIES)
    return out


# speedup vs baseline: 1.6452x; 1.2300x over previous
import functools

import jax
import jax.numpy as jnp
import numpy as np
from jax import lax
from jax.experimental import pallas as pl
from jax.experimental.pallas import tpu as pltpu
from jax.experimental.pallas import tpu_sc as plsc

_ALL_SPECIES = np.array([1, 6, 7, 8], dtype=np.int32)
_N_SPECIES = 4
_R_CUT = 5.0
_AVG_NEIGH = 40.0
_Z = [
    np.array([3.14159265, 6.28318531, 9.42477796, 12.56637061, 15.70796327,
              18.84955592, 21.99114858, 25.13274123]),
    np.array([4.49340946, 7.72525184, 10.90412166, 14.06619391, 17.22075527,
              20.37130296, 23.51945250]),
    np.array([5.76345920, 9.09501133, 12.32294097, 15.51460301, 18.68903640,
              21.85387410]),
    np.array([6.98793200, 10.41711855, 13.69802315, 16.92362129, 20.12180617]),
]
_NL = [8, 7, 6, 5]
_NF = 26


def _feat_body(zcol_ref, d2_ref, out_ref):
    d2 = d2_ref[0]
    r = jnp.sqrt(d2 + 1e-12)
    x = zcol_ref[...] * r / _R_CUT
    s, c = jnp.sin(x), jnp.cos(x)
    row = 0
    for l in range(4):
        sl = slice(row, row + _NL[l])
        xs, ss, cc = x[sl], s[sl], c[sl]
        if l == 0:
            rb = ss / xs
        elif l == 1:
            rb = ss / (xs * xs) - cc / xs
        elif l == 2:
            x2 = xs * xs
            x3 = x2 * xs
            rb = (3.0 / x3 - 1.0 / xs) * ss - 3.0 * cc / x2
        else:
            x2 = xs * xs
            x3 = x2 * xs
            x4 = x2 * x2
            rb = (15.0 / x4 - 6.0 / x2) * ss - (15.0 / x3 - 1.0 / xs) * cc
        norm = (1.0 / _AVG_NEIGH ** 0.75) if l == 0 else (1.0 / float(np.sqrt(_AVG_NEIGH)))
        out_ref[sl, :] = rb * np.float32(norm)
        row += _NL[l]


def _features_planes(d2):
    E = d2.shape[0]
    BF = 2560
    assert E % BF == 0
    g = E // BF
    d2v = d2.reshape(g, 1, BF)
    zcol = jnp.asarray(np.concatenate(_Z).astype(np.float32).reshape(_NF, 1))
    return pl.pallas_call(
        _feat_body,
        grid=(g,),
        in_specs=[
            pl.BlockSpec((_NF, 1), lambda i: (0, 0)),
            pl.BlockSpec((1, 1, BF), lambda i: (i, 0, 0)),
        ],
        out_specs=pl.BlockSpec((_NF, BF), lambda i: (0, i)),
        out_shape=jax.ShapeDtypeStruct((_NF, E), jnp.float32),
    )(zcol, d2v)


_EPAD = 1638400
_ROWS = 400000
_DUMP = 4096
_ROWS_PAD = 409600
_CHUNK_E = 4096
_TILE_E = 102400
_PASSES = (3, 3, 3, 3, 1)


def _sc_scatter_body(feat_hbm, idx_hbm, zeros_hbm, dens_hbm,
                     idx_v, f_v, z_v, a0, a1, a2):
    planes = [a0, a1, a2]
    cid = lax.axis_index("c")
    sid = lax.axis_index("s")
    pltpu.sync_copy(zeros_hbm, z_v)
    dslice = pl.ds(sid * 25600, 25600)
    pbase = 0
    for npl_, _pass in zip(_PASSES, range(len(_PASSES))):
        for u in range(npl_):
            pltpu.sync_copy(z_v, planes[u].at[pl.ds(sid * 25000, 25000)])
        plsc.subcore_barrier()

        def chunk_body(it, carry, pbase=pbase, npl_=npl_):
            e0_ = sid * _TILE_E + it * _CHUNK_E
            pltpu.sync_copy(idx_hbm.at[pl.ds(e0_, _CHUNK_E)], idx_v)
            for u in range(npl_):
                jj = cid * 13 + pbase + u
                pltpu.sync_copy(feat_hbm.at[jj, 0, pl.ds(e0_, _CHUNK_E)], f_v)
                pltpu.sync_copy(f_v, planes[u].at[idx_v], add=True)
            return carry

        lax.fori_loop(0, _TILE_E // _CHUNK_E, chunk_body, 0)
        plsc.subcore_barrier()
        for u in range(npl_):
            jj = cid * 13 + pbase + u
            pltpu.sync_copy(planes[u].at[dslice], dens_hbm.at[jj, 0, dslice])
        plsc.subcore_barrier()
        pbase += npl_


def _sc_scatter(feats3, idx2, zeros):
    mesh = plsc.VectorSubcoreMesh(core_axis_name="c", subcore_axis_name="s")
    f = pl.kernel(
        _sc_scatter_body,
        mesh=mesh,
        out_type=jax.ShapeDtypeStruct((_NF, 1, _ROWS_PAD), jnp.float32),
        scratch_types=[
            pltpu.VMEM((_CHUNK_E,), jnp.int32),
            pltpu.VMEM((_CHUNK_E,), jnp.float32),
            pltpu.VMEM((25000,), jnp.float32),
        ] + [pltpu.VMEM_SHARED((_ROWS_PAD,), jnp.float32) for _ in range(3)],
    )
    return f(feats3, idx2, zeros)


def kernel(positions, cell, edge_shifts, numbers, edge_index):
    N = positions.shape[0]
    E = edge_index.shape[1]
    e0 = edge_index[0]
    e1 = edge_index[1]
    vec = positions[e1] - positions[e0] + edge_shifts @ cell
    d2 = jnp.sum(vec * vec, axis=-1)
    pad = _EPAD - E
    d2p = jnp.concatenate([d2, jnp.full((pad,), 1.0, jnp.float32)])
    feats = _features_planes(d2p)

    species_idx = jnp.searchsorted(jnp.asarray(_ALL_SPECIES), numbers)
    dens_idx = e0 * _N_SPECIES + species_idx[e1]
    dump = _ROWS + (jnp.arange(pad, dtype=jnp.int32) % _DUMP)
    idx2 = jnp.concatenate([dens_idx.astype(jnp.int32), dump])
    feats3 = feats.reshape(_NF, 1, _EPAD)
    zeros = jnp.zeros((25000,), jnp.float32)
    dens = _sc_scatter(feats3, idx2, zeros)
    dens = dens.reshape(_NF, _ROWS_PAD)[:, :_ROWS]
    out = dens.reshape(_NF, N, _N_SPECIES).transpose(1, 0, 2).reshape(N, _NF * _N_SPECIES)
    return out


# speedup vs baseline: 13.7547x; 8.3605x over previous
import functools

import jax
import jax.numpy as jnp
import numpy as np
from jax import lax
from jax.experimental import pallas as pl
from jax.experimental.pallas import tpu as pltpu
from jax.experimental.pallas import tpu_sc as plsc

_ALL_SPECIES = np.array([1, 6, 7, 8], dtype=np.int32)
_N_SPECIES = 4
_R_CUT = 5.0
_AVG_NEIGH = 40.0
_Z = [
    np.array([3.14159265, 6.28318531, 9.42477796, 12.56637061, 15.70796327,
              18.84955592, 21.99114858, 25.13274123]),
    np.array([4.49340946, 7.72525184, 10.90412166, 14.06619391, 17.22075527,
              20.37130296, 23.51945250]),
    np.array([5.76345920, 9.09501133, 12.32294097, 15.51460301, 18.68903640,
              21.85387410]),
    np.array([6.98793200, 10.41711855, 13.69802315, 16.92362129, 20.12180617]),
]
_NL = [8, 7, 6, 5]
_NF = 26


def _feat_body(zcol_ref, d2_ref, out_ref):
    d2 = d2_ref[0]
    r = jnp.sqrt(d2 + 1e-12)
    x = zcol_ref[...] * r / _R_CUT
    s, c = jnp.sin(x), jnp.cos(x)
    row = 0
    for l in range(4):
        sl = slice(row, row + _NL[l])
        xs, ss, cc = x[sl], s[sl], c[sl]
        if l == 0:
            rb = ss / xs
        elif l == 1:
            rb = ss / (xs * xs) - cc / xs
        elif l == 2:
            x2 = xs * xs
            x3 = x2 * xs
            rb = (3.0 / x3 - 1.0 / xs) * ss - 3.0 * cc / x2
        else:
            x2 = xs * xs
            x3 = x2 * xs
            x4 = x2 * x2
            rb = (15.0 / x4 - 6.0 / x2) * ss - (15.0 / x3 - 1.0 / xs) * cc
        norm = (1.0 / _AVG_NEIGH ** 0.75) if l == 0 else (1.0 / float(np.sqrt(_AVG_NEIGH)))
        out_ref[sl, :] = rb * np.float32(norm)
        row += _NL[l]


def _features_planes(d2):
    E = d2.shape[0]
    BF = 2560
    assert E % BF == 0
    g = E // BF
    d2v = d2.reshape(g, 1, BF)
    zcol = jnp.asarray(np.concatenate(_Z).astype(np.float32).reshape(_NF, 1))
    return pl.pallas_call(
        _feat_body,
        grid=(g,),
        in_specs=[
            pl.BlockSpec((_NF, 1), lambda i: (0, 0)),
            pl.BlockSpec((1, 1, BF), lambda i: (i, 0, 0)),
        ],
        out_specs=pl.BlockSpec((_NF, BF), lambda i: (0, i)),
        out_shape=jax.ShapeDtypeStruct((_NF, E), jnp.float32),
    )(zcol, d2v)


_GCHUNK = 2048
_GTILE = 51200


def _sc_gather_body(pos3_hbm, sp_hbm, e0_hbm, e1_hbm, d2_hbm, dens_hbm,
                    i0_v, i1_v, ax_v, ay_v, az_v, bx_v, by_v, bz_v, bs_v,
                    d2b_v, densb_v, px_s, py_s, pz_s, sp_s):
    cid = lax.axis_index("c")
    sid = lax.axis_index("s")
    wid = cid * 16 + sid
    for p, dst in ((0, px_s), (1, py_s), (2, pz_s)):
        @pl.when(sid == p)
        def _(p=p, dst=dst):
            pltpu.sync_copy(pos3_hbm.at[p, 0], dst)

    @pl.when(sid == 3)
    def _():
        pltpu.sync_copy(sp_hbm, sp_s)
    plsc.subcore_barrier()

    def chunk_body(it, carry):
        base = wid * _GTILE + it * _GCHUNK
        pltpu.sync_copy(e0_hbm.at[pl.ds(base, _GCHUNK)], i0_v)
        pltpu.sync_copy(e1_hbm.at[pl.ds(base, _GCHUNK)], i1_v)
        pltpu.sync_copy(px_s.at[i0_v], ax_v)
        pltpu.sync_copy(py_s.at[i0_v], ay_v)
        pltpu.sync_copy(pz_s.at[i0_v], az_v)
        pltpu.sync_copy(px_s.at[i1_v], bx_v)
        pltpu.sync_copy(py_s.at[i1_v], by_v)
        pltpu.sync_copy(pz_s.at[i1_v], bz_v)
        pltpu.sync_copy(sp_s.at[i1_v], bs_v)

        def grp(g, c):
            s16 = pl.ds(g * 16, 16)
            dx = ax_v[s16] - bx_v[s16]
            dy = ay_v[s16] - by_v[s16]
            dz = az_v[s16] - bz_v[s16]
            d2b_v[s16] = dx * dx + dy * dy + dz * dz
            gidx = base + g * 16 + lax.iota(jnp.int32, 16)
            dens = i0_v[s16] * 4 + bs_v[s16]
            densb_v[s16] = jnp.where(gidx < 1600000, dens,
                                     _ROWS + (gidx & (_DUMP - 1)))
            return c

        lax.fori_loop(0, _GCHUNK // 16, grp, 0)
        pltpu.sync_copy(d2b_v, d2_hbm.at[pl.ds(base, _GCHUNK)])
        pltpu.sync_copy(densb_v, dens_hbm.at[pl.ds(base, _GCHUNK)])
        return carry

    lax.fori_loop(0, _GTILE // _GCHUNK, chunk_body, 0)


def _sc_gather(pos3, sp, e0p, e1p):
    mesh = plsc.VectorSubcoreMesh(core_axis_name="c", subcore_axis_name="s")
    f = pl.kernel(
        _sc_gather_body,
        mesh=mesh,
        out_type=(jax.ShapeDtypeStruct((_EPAD,), jnp.float32),
                  jax.ShapeDtypeStruct((_EPAD,), jnp.int32)),
        scratch_types=[
            pltpu.VMEM((_GCHUNK,), jnp.int32),
            pltpu.VMEM((_GCHUNK,), jnp.int32),
        ] + [pltpu.VMEM((_GCHUNK,), jnp.float32) for _ in range(6)] + [
            pltpu.VMEM((_GCHUNK,), jnp.int32),
            pltpu.VMEM((_GCHUNK,), jnp.float32),
            pltpu.VMEM((_GCHUNK,), jnp.int32),
            pltpu.VMEM_SHARED((100000,), jnp.float32),
            pltpu.VMEM_SHARED((100000,), jnp.float32),
            pltpu.VMEM_SHARED((100000,), jnp.float32),
            pltpu.VMEM_SHARED((100000,), jnp.int32),
        ],
    )
    return f(pos3, sp, e0p, e1p)


_EPAD = 1638400
_ROWS = 400000
_DUMP = 4096
_ROWS_PAD = 409600
_CHUNK_E = 4096
_TILE_E = 102400
_PASSES = (3, 3, 3, 3, 1)


def _sc_scatter_body(feat_hbm, idx_hbm, zeros_hbm, dens_hbm,
                     idx_v, f_v, z_v, a0, a1, a2):
    planes = [a0, a1, a2]
    cid = lax.axis_index("c")
    sid = lax.axis_index("s")
    pltpu.sync_copy(zeros_hbm, z_v)
    dslice = pl.ds(sid * 25600, 25600)
    pbase = 0
    for npl_, _pass in zip(_PASSES, range(len(_PASSES))):
        for u in range(npl_):
            pltpu.sync_copy(z_v, planes[u].at[pl.ds(sid * 25000, 25000)])
        plsc.subcore_barrier()

        def chunk_body(it, carry, pbase=pbase, npl_=npl_):
            e0_ = sid * _TILE_E + it * _CHUNK_E
            pltpu.sync_copy(idx_hbm.at[pl.ds(e0_, _CHUNK_E)], idx_v)
            for u in range(npl_):
                jj = cid * 13 + pbase + u
                pltpu.sync_copy(feat_hbm.at[jj, 0, pl.ds(e0_, _CHUNK_E)], f_v)
                pltpu.sync_copy(f_v, planes[u].at[idx_v], add=True)
            return carry

        lax.fori_loop(0, _TILE_E // _CHUNK_E, chunk_body, 0)
        plsc.subcore_barrier()
        for u in range(npl_):
            jj = cid * 13 + pbase + u
            pltpu.sync_copy(planes[u].at[dslice], dens_hbm.at[jj, 0, dslice])
        plsc.subcore_barrier()
        pbase += npl_


def _sc_scatter(feats3, idx2, zeros):
    mesh = plsc.VectorSubcoreMesh(core_axis_name="c", subcore_axis_name="s")
    f = pl.kernel(
        _sc_scatter_body,
        mesh=mesh,
        out_type=jax.ShapeDtypeStruct((_NF, 1, _ROWS_PAD), jnp.float32),
        scratch_types=[
            pltpu.VMEM((_CHUNK_E,), jnp.int32),
            pltpu.VMEM((_CHUNK_E,), jnp.float32),
            pltpu.VMEM((25000,), jnp.float32),
        ] + [pltpu.VMEM_SHARED((_ROWS_PAD,), jnp.float32) for _ in range(3)],
    )
    return f(feats3, idx2, zeros)


def kernel(positions, cell, edge_shifts, numbers, edge_index):
    N = positions.shape[0]
    E = edge_index.shape[1]
    e0 = edge_index[0]
    e1 = edge_index[1]
    pad = _EPAD - E
    zpad = jnp.zeros((pad,), jnp.int32)
    e0p = jnp.concatenate([e0.astype(jnp.int32), zpad])
    e1p = jnp.concatenate([e1.astype(jnp.int32), zpad])
    species_idx = jnp.searchsorted(jnp.asarray(_ALL_SPECIES), numbers).astype(jnp.int32)
    pos3 = positions.T.reshape(3, 1, N)
    d2p, idx2 = _sc_gather(pos3, species_idx, e0p, e1p)
    feats = _features_planes(d2p)
    feats3 = feats.reshape(_NF, 1, _EPAD)
    zeros = jnp.zeros((25000,), jnp.float32)
    dens = _sc_scatter(feats3, idx2, zeros)
    dens = dens.reshape(_NF, _ROWS_PAD)[:, :_ROWS]
    out = dens.reshape(_NF, N, _N_SPECIES).transpose(1, 0, 2).reshape(N, _NF * _N_SPECIES)
    return out


# speedup vs baseline: 14.4229x; 1.0486x over previous
import functools

import jax
import jax.numpy as jnp
import numpy as np
from jax import lax
from jax.experimental import pallas as pl
from jax.experimental.pallas import tpu as pltpu
from jax.experimental.pallas import tpu_sc as plsc

_ALL_SPECIES = np.array([1, 6, 7, 8], dtype=np.int32)
_N_SPECIES = 4
_R_CUT = 5.0
_AVG_NEIGH = 40.0
_Z = [
    np.array([3.14159265, 6.28318531, 9.42477796, 12.56637061, 15.70796327,
              18.84955592, 21.99114858, 25.13274123]),
    np.array([4.49340946, 7.72525184, 10.90412166, 14.06619391, 17.22075527,
              20.37130296, 23.51945250]),
    np.array([5.76345920, 9.09501133, 12.32294097, 15.51460301, 18.68903640,
              21.85387410]),
    np.array([6.98793200, 10.41711855, 13.69802315, 16.92362129, 20.12180617]),
]
_NL = [8, 7, 6, 5]
_NF = 26


def _feat_body(zcol_ref, d2_ref, out_ref):
    d2 = d2_ref[0]
    r = jnp.sqrt(d2 + 1e-12)
    zc = zcol_ref[...]
    x = zc * r / _R_CUT
    s, c = jnp.sin(x), jnp.cos(x)
    u = 1.0 / x

    r0 = jnp.sqrt(jnp.full((1, 1), 0.0, jnp.float32) + 1e-12)
    x0 = zc * r0 / _R_CUT
    s0, c0 = jnp.sin(x0), jnp.cos(x0)
    selfmask = (d2 == 0.0)

    row = 0
    for l in range(4):
        sl = slice(row, row + _NL[l])
        ss, cc, uu = s[sl], c[sl], u[sl]
        if l == 0:
            rb = ss * uu
            g0 = s0[sl] / x0[sl]
        elif l == 1:
            rb = (ss * uu - cc) * uu
            g0 = s0[sl] / (x0[sl] * x0[sl]) - c0[sl] / x0[sl]
        elif l == 2:
            u2 = uu * uu
            rb = (3.0 * u2 - 1.0) * (ss * uu) - 3.0 * cc * u2
            y = x0[sl]
            y2 = y * y
            y3 = y2 * y
            g0 = (3.0 / y3 - 1.0 / y) * s0[sl] - 3.0 * c0[sl] / y2
        else:
            u2 = uu * uu
            rb = (15.0 * u2 - 6.0) * u2 * ss + (1.0 - 15.0 * u2) * uu * cc
            y = x0[sl]
            y2 = y * y
            y3 = y2 * y
            y4 = y2 * y2
            g0 = (15.0 / y4 - 6.0 / y2) * s0[sl] - (15.0 / y3 - 1.0 / y) * c0[sl]
        rb = jnp.where(selfmask, g0, rb)
        norm = (1.0 / _AVG_NEIGH ** 0.75) if l == 0 else (1.0 / float(np.sqrt(_AVG_NEIGH)))
        out_ref[sl, :] = rb * np.float32(norm)
        row += _NL[l]


def _features_planes(d2):
    E = d2.shape[0]
    BF = 2560
    assert E % BF == 0
    g = E // BF
    d2v = d2.reshape(g, 1, BF)
    zcol = jnp.asarray(np.concatenate(_Z).astype(np.float32).reshape(_NF, 1))
    return pl.pallas_call(
        _feat_body,
        grid=(g,),
        in_specs=[
            pl.BlockSpec((_NF, 1), lambda i: (0, 0)),
            pl.BlockSpec((1, 1, BF), lambda i: (i, 0, 0)),
        ],
        out_specs=pl.BlockSpec((_NF, BF), lambda i: (0, i)),
        out_shape=jax.ShapeDtypeStruct((_NF, E), jnp.float32),
    )(zcol, d2v)


_GCHUNK = 2048
_GTILE = 51200


def _sc_gather_body(pos3_hbm, sp_hbm, e0_hbm, e1_hbm, d2_hbm, dens_hbm,
                    i0_v, i1_v, ax_v, ay_v, az_v, bx_v, by_v, bz_v, bs_v,
                    d2b_v, densb_v, px_s, py_s, pz_s, sp_s):
    cid = lax.axis_index("c")
    sid = lax.axis_index("s")
    wid = cid * 16 + sid
    for p, dst in ((0, px_s), (1, py_s), (2, pz_s)):
        @pl.when(sid == p)
        def _(p=p, dst=dst):
            pltpu.sync_copy(pos3_hbm.at[p, 0], dst)

    @pl.when(sid == 3)
    def _():
        pltpu.sync_copy(sp_hbm, sp_s)
    plsc.subcore_barrier()

    def chunk_body(it, carry):
        base = wid * _GTILE + it * _GCHUNK
        pltpu.sync_copy(e0_hbm.at[pl.ds(base, _GCHUNK)], i0_v)
        pltpu.sync_copy(e1_hbm.at[pl.ds(base, _GCHUNK)], i1_v)
        pltpu.sync_copy(px_s.at[i0_v], ax_v)
        pltpu.sync_copy(py_s.at[i0_v], ay_v)
        pltpu.sync_copy(pz_s.at[i0_v], az_v)
        pltpu.sync_copy(px_s.at[i1_v], bx_v)
        pltpu.sync_copy(py_s.at[i1_v], by_v)
        pltpu.sync_copy(pz_s.at[i1_v], bz_v)
        pltpu.sync_copy(sp_s.at[i1_v], bs_v)

        def grp(g, c):
            s16 = pl.ds(g * 16, 16)
            dx = ax_v[s16] - bx_v[s16]
            dy = ay_v[s16] - by_v[s16]
            dz = az_v[s16] - bz_v[s16]
            d2b_v[s16] = dx * dx + dy * dy + dz * dz
            gidx = base + g * 16 + lax.iota(jnp.int32, 16)
            dens = i0_v[s16] * 4 + bs_v[s16]
            densb_v[s16] = jnp.where(gidx < 1600000, dens,
                                     _ROWS + (gidx & (_DUMP - 1)))
            return c

        lax.fori_loop(0, _GCHUNK // 16, grp, 0)
        pltpu.sync_copy(d2b_v, d2_hbm.at[pl.ds(base, _GCHUNK)])
        pltpu.sync_copy(densb_v, dens_hbm.at[pl.ds(base, _GCHUNK)])
        return carry

    lax.fori_loop(0, _GTILE // _GCHUNK, chunk_body, 0)


def _sc_gather(pos3, sp, e0p, e1p):
    mesh = plsc.VectorSubcoreMesh(core_axis_name="c", subcore_axis_name="s")
    f = pl.kernel(
        _sc_gather_body,
        mesh=mesh,
        out_type=(jax.ShapeDtypeStruct((_EPAD,), jnp.float32),
                  jax.ShapeDtypeStruct((_EPAD,), jnp.int32)),
        scratch_types=[
            pltpu.VMEM((_GCHUNK,), jnp.int32),
            pltpu.VMEM((_GCHUNK,), jnp.int32),
        ] + [pltpu.VMEM((_GCHUNK,), jnp.float32) for _ in range(6)] + [
            pltpu.VMEM((_GCHUNK,), jnp.int32),
            pltpu.VMEM((_GCHUNK,), jnp.float32),
            pltpu.VMEM((_GCHUNK,), jnp.int32),
            pltpu.VMEM_SHARED((100000,), jnp.float32),
            pltpu.VMEM_SHARED((100000,), jnp.float32),
            pltpu.VMEM_SHARED((100000,), jnp.float32),
            pltpu.VMEM_SHARED((100000,), jnp.int32),
        ],
    )
    return f(pos3, sp, e0p, e1p)


_EPAD = 1638400
_ROWS = 400000
_DUMP = 4096
_ROWS_PAD = 409600
_CHUNK_E = 4096
_TILE_E = 102400
_PASSES = (4, 4, 4, 1)


def _sc_scatter_body(feat_hbm, idx_hbm, zeros_hbm, dens_hbm,
                     idx_v, f_v, z_v, a0, a1, a2, a3):
    planes = [a0, a1, a2, a3]
    cid = lax.axis_index("c")
    sid = lax.axis_index("s")
    pltpu.sync_copy(zeros_hbm, z_v)
    dslice = pl.ds(sid * 25600, 25600)
    zoff = sid * 25600
    pbase = 0
    for npl_, _pass in zip(_PASSES, range(len(_PASSES))):
        for u in range(npl_):
            for zk in range(4):
                pltpu.sync_copy(z_v, planes[u].at[pl.ds(zoff + zk * 6400, 6400)])
        plsc.subcore_barrier()

        def chunk_body(it, carry, pbase=pbase, npl_=npl_):
            e0_ = sid * _TILE_E + it * _CHUNK_E
            pltpu.sync_copy(idx_hbm.at[pl.ds(e0_, _CHUNK_E)], idx_v)
            for u in range(npl_):
                jj = cid * 13 + pbase + u
                pltpu.sync_copy(feat_hbm.at[jj, 0, pl.ds(e0_, _CHUNK_E)], f_v)
                pltpu.sync_copy(f_v, planes[u].at[idx_v], add=True)
            return carry

        lax.fori_loop(0, _TILE_E // _CHUNK_E, chunk_body, 0)
        plsc.subcore_barrier()
        for u in range(npl_):
            jj = cid * 13 + pbase + u
            pltpu.sync_copy(planes[u].at[dslice], dens_hbm.at[jj, 0, dslice])
        plsc.subcore_barrier()
        pbase += npl_


def _sc_scatter(feats3, idx2, zeros):
    mesh = plsc.VectorSubcoreMesh(core_axis_name="c", subcore_axis_name="s")
    f = pl.kernel(
        _sc_scatter_body,
        mesh=mesh,
        out_type=jax.ShapeDtypeStruct((_NF, 1, _ROWS_PAD), jnp.float32),
        scratch_types=[
            pltpu.VMEM((_CHUNK_E,), jnp.int32),
            pltpu.VMEM((_CHUNK_E,), jnp.float32),
            pltpu.VMEM((6400,), jnp.float32),
        ] + [pltpu.VMEM_SHARED((_ROWS_PAD,), jnp.float32) for _ in range(4)],
    )
    return f(feats3, idx2, zeros)


def kernel(positions, cell, edge_shifts, numbers, edge_index):
    N = positions.shape[0]
    E = edge_index.shape[1]
    e0 = edge_index[0]
    e1 = edge_index[1]
    pad = _EPAD - E
    spread = jnp.arange(pad, dtype=jnp.int32) * 13 % N
    e0p = jnp.concatenate([e0.astype(jnp.int32), spread])
    e1p = jnp.concatenate([e1.astype(jnp.int32), spread])
    species_idx = jnp.searchsorted(jnp.asarray(_ALL_SPECIES), numbers).astype(jnp.int32)
    pos3 = positions.T.reshape(3, 1, N)
    d2p, idx2 = _sc_gather(pos3, species_idx, e0p, e1p)
    feats = _features_planes(d2p)
    feats3 = feats.reshape(_NF, 1, _EPAD)
    zeros = jnp.zeros((6400,), jnp.float32)
    dens = _sc_scatter(feats3, idx2, zeros)
    dens = dens.reshape(_NF, _ROWS_PAD)[:, :_ROWS]
    out = dens.reshape(_NF, N, _N_SPECIES).transpose(1, 0, 2).reshape(N, _NF * _N_SPECIES)
    return out
